# Initial kernel scaffold; baseline (speedup 1.0000x reference)
import functools

import jax
import jax.numpy as jnp
from jax import lax
from jax.experimental import pallas as pl
from jax.experimental.pallas import tpu as pltpu
from jax.experimental.pallas import tpu_sc as plsc

VOCAB = 1000000
DIM = 64
BATCH = 16384
HIST = 50

B = BATCH * HIST
NW = 32
BPW = B // NW
IDXW = 128
ROWS_PW = BPW // IDXW
CHUNK = 512
GPC = CHUNK // IDXW
NCHUNK = BPW // CHUNK


def _embed_body(idx_hbm, table_hbm, out_hbm, idx_v, rows_v, gsem):
    cid = lax.axis_index("c")
    sid = lax.axis_index("s")
    wid = sid * 2 + cid
    pltpu.sync_copy(idx_hbm.at[pl.ds(wid * ROWS_PW, ROWS_PW)], idx_v)
    out_base = wid * BPW

    def chunk_body(c, carry):
        cps = []
        for j in range(GPC):
            cp = pltpu.async_copy(
                table_hbm.at[idx_v.at[c * GPC + j]],
                rows_v.at[pl.ds(j * IDXW, IDXW)],
                gsem,
            )
            cps.append(cp)
        for cp in cps:
            cp.wait()
        pltpu.sync_copy(rows_v, out_hbm.at[pl.ds(out_base + c * CHUNK, CHUNK)])
        return carry

    lax.fori_loop(0, NCHUNK, chunk_body, 0)


@functools.partial(jax.jit, static_argnames=())
def kernel(input_, table):
    idx = input_.reshape(NW * ROWS_PW, IDXW).astype(jnp.int32)
    mesh = plsc.VectorSubcoreMesh(core_axis_name="c", subcore_axis_name="s")
    out = pl.kernel(
        _embed_body,
        out_type=jax.ShapeDtypeStruct((B, DIM), jnp.float32),
        mesh=mesh,
        scratch_types=[
            pltpu.VMEM((ROWS_PW, IDXW), jnp.int32),
            pltpu.VMEM((CHUNK, DIM), jnp.float32),
            pltpu.SemaphoreType.DMA,
        ],
    )(idx, table)
    return out.reshape(BATCH, HIST, DIM)


# speedup vs baseline: 1.8306x; 1.8306x over previous
import functools

import jax
import jax.numpy as jnp
from jax import lax
from jax.experimental import pallas as pl
from jax.experimental.pallas import tpu as pltpu
from jax.experimental.pallas import tpu_sc as plsc

VOCAB = 1000000
DIM = 64
BATCH = 16384
HIST = 50

B = BATCH * HIST
NW = 32
BPW = B // NW
IDXW = 128
ROWS_PW = BPW // IDXW
CHUNK = 512
GPC = CHUNK // IDXW
NCHUNK = BPW // CHUNK


def _embed_body(idx_hbm, table_hbm, out_hbm, idx_v, rows_v, gsem):
    cid = lax.axis_index("c")
    sid = lax.axis_index("s")
    wid = sid * 2 + cid
    pltpu.sync_copy(idx_hbm.at[pl.ds(wid * ROWS_PW, ROWS_PW)], idx_v)
    out_base = wid * BPW

    def chunk_body(c, carry):
        cps = []
        for j in range(GPC):
            cp = pltpu.async_copy(
                table_hbm.at[idx_v.at[c * GPC + j]],
                rows_v.at[pl.ds(j * IDXW, IDXW)],
                gsem,
            )
            cps.append(cp)
        for cp in cps:
            cp.wait()
        pltpu.sync_copy(rows_v, out_hbm.at[pl.ds(out_base + c * CHUNK, CHUNK)])
        return carry

    lax.fori_loop(0, NCHUNK, chunk_body, 0)


@functools.partial(jax.jit, static_argnames=())
def kernel(input_, table):
    idx = input_.reshape(NW * ROWS_PW, IDXW).astype(jnp.int32)
    mesh = plsc.VectorSubcoreMesh(core_axis_name="c", subcore_axis_name="s")
    out = pl.kernel(
        _embed_body,
        out_type=jax.ShapeDtypeStruct((B, DIM), jnp.float32),
        mesh=mesh,
        scratch_types=[
            pltpu.VMEM((ROWS_PW, IDXW), jnp.int32),
            pltpu.VMEM((CHUNK, DIM), jnp.float32),
            pltpu.SemaphoreType.DMA,
        ],
        compiler_params=pltpu.CompilerParams(use_tc_tiling_on_sc=False),
    )(idx, table)
    return out.reshape(BATCH, HIST, DIM)


# speedup vs baseline: 1.8702x; 1.0217x over previous
import functools

import jax
import jax.numpy as jnp
from jax import lax
from jax.experimental import pallas as pl
from jax.experimental.pallas import tpu as pltpu
from jax.experimental.pallas import tpu_sc as plsc

VOCAB = 1000000
DIM = 64
BATCH = 16384
HIST = 50

B = BATCH * HIST
NW = 32
BPW = B // NW
IDXW = 128
ROWS_PW = BPW // IDXW
CHUNK = 256
GPC = CHUNK // IDXW
NCHUNK = BPW // CHUNK
NBUF = 4
T = NCHUNK // NBUF


def _embed_body(idx_hbm, table_hbm, out_hbm, idx_v, rows_v,
                gs0, gs1, gs2, gs3, os0, os1, os2, os3):
    gsems = (gs0, gs1, gs2, gs3)
    osems = (os0, os1, os2, os3)
    cid = lax.axis_index("c")
    sid = lax.axis_index("s")
    wid = sid * 2 + cid
    pltpu.sync_copy(idx_hbm.at[pl.ds(wid * ROWS_PW, ROWS_PW)], idx_v)
    out_base = wid * BPW

    def gathers(c, b):
        for j in range(GPC):
            pltpu.async_copy(
                table_hbm.at[idx_v.at[c * GPC + j]],
                rows_v.at[b].at[pl.ds(j * IDXW, IDXW)],
                gsems[b],
            )

    def wait_gathers(c, b):
        for j in range(GPC):
            pltpu.make_async_copy(
                table_hbm.at[idx_v.at[c * GPC + j]],
                rows_v.at[b].at[pl.ds(j * IDXW, IDXW)],
                gsems[b],
            ).wait()

    def out_copy(c, b):
        return pltpu.make_async_copy(
            rows_v.at[b],
            out_hbm.at[pl.ds(out_base + c * CHUNK, CHUNK)],
            osems[b],
        )

    for b in range(NBUF):
        gathers(b, b)

    def ring_body(t, carry):
        c0 = t * NBUF
        for b in range(NBUF):
            wait_gathers(c0 + b, b)
            out_copy(c0 + b, b).start()

        @pl.when(t < T - 1)
        def _():
            for b in range(NBUF):
                out_copy(c0 + b, b).wait()
                gathers(c0 + NBUF + b, b)

        return carry

    lax.fori_loop(0, T, ring_body, 0)

    for b in range(NBUF):
        out_copy(NCHUNK - NBUF + b, b).wait()


@functools.partial(jax.jit, static_argnames=())
def kernel(input_, table):
    idx = input_.reshape(NW * ROWS_PW, IDXW).astype(jnp.int32)
    mesh = plsc.VectorSubcoreMesh(core_axis_name="c", subcore_axis_name="s")
    out = pl.kernel(
        _embed_body,
        out_type=jax.ShapeDtypeStruct((B, DIM), jnp.float32),
        mesh=mesh,
        scratch_types=[
            pltpu.VMEM((ROWS_PW, IDXW), jnp.int32),
            pltpu.VMEM((NBUF, CHUNK, DIM), jnp.float32),
        ] + [pltpu.SemaphoreType.DMA] * (2 * NBUF),
        compiler_params=pltpu.CompilerParams(use_tc_tiling_on_sc=False),
    )(idx, table)
    return out.reshape(BATCH, HIST, DIM)
